# baseline (speedup 1.0000x reference)
import functools

import jax
import jax.numpy as jnp
import numpy as np
from jax.experimental import pallas as pl

N = 10000
E = 160000
D_FEAT = 256
HID = 256
HEADS = 4
LAYERS = 2
OUT_DIM = 1000
EOS_TOK = OUT_DIM + 1


def _mm_kernel(x_ref, w_ref, b_ref, o_ref):
    o_ref[...] = (
        jnp.dot(x_ref[...], w_ref[...], preferred_element_type=jnp.float32)
        + b_ref[...]
    )


def matmul_bias(x, w, b, block_rows=1000):
    n, din = x.shape
    dout = w.shape[1]
    grid = (pl.cdiv(n, block_rows),)
    return pl.pallas_call(
        _mm_kernel,
        grid=grid,
        in_specs=[
            pl.BlockSpec((block_rows, din), lambda i: (i, 0)),
            pl.BlockSpec((din, dout), lambda i: (0, 0)),
            pl.BlockSpec((dout,), lambda i: (0,)),
        ],
        out_specs=pl.BlockSpec((block_rows, dout), lambda i: (i, 0)),
        out_shape=jax.ShapeDtypeStruct((n, dout), jnp.float32),
    )(x, w, b)


def _lin(p, x):
    return matmul_bias(x, p["W"], p["b"])


def _sage(p, x_src, x_dst, ei, num_dst):
    src, dst = ei[0], ei[1]
    s = jax.ops.segment_sum(x_src[src], dst, num_segments=num_dst)
    cnt = jax.ops.segment_sum(jnp.ones((ei.shape[1],), jnp.float32), dst,
                              num_segments=num_dst)
    agg = s / jnp.maximum(cnt, 1.0)[:, None]
    return _lin(p["l"], agg) + matmul_bias(x_dst, p["Wr"],
                                           jnp.zeros((p["Wr"].shape[1],), jnp.float32))


def _tconv(p, x_src, x_dst, ei, heads, dout, num_dst):
    src, dst = ei[0], ei[1]
    q = _lin(p["q"], x_dst).reshape(-1, heads, dout)
    k = _lin(p["k"], x_src).reshape(-1, heads, dout)
    v = _lin(p["v"], x_src).reshape(-1, heads, dout)
    logits = (q[dst] * k[src]).sum(-1) / np.sqrt(dout)
    m = jax.ops.segment_max(logits, dst, num_segments=num_dst)
    ex = jnp.exp(logits - m[dst])
    den = jax.ops.segment_sum(ex, dst, num_segments=num_dst)
    alpha = ex / (den[dst] + 1e-16)
    out = jax.ops.segment_sum(v[src] * alpha[:, :, None], dst,
                              num_segments=num_dst)
    return out.reshape(num_dst, heads * dout) + _lin(p["skip"], x_dst)


def _gln(p, x):
    mu = jnp.mean(x)
    var = jnp.mean((x - mu) ** 2)
    return (x - mu) / jnp.sqrt(var + 1e-5) * p["g"] + p["b"]


def _attention(p, x, ei, n):
    h = _tconv(p["att"], x, x, ei, HEADS, HID, n)
    return _gln(p["norm"], _lin(p["cat"], h))


def _ffw(p, x, ei, n):
    return _gln(p["norm"], _sage(p["sage"], x, x, ei, n))


def kernel(ast_x, llc_x, params, ast_edge_index, llc_edge_index):
    x = _sage(params["enc"]["embed"], llc_x, llc_x, llc_edge_index, N)
    for u in params["enc"]["units"]:
        x = _attention(u["att"], x, llc_edge_index, N)
        x = _ffw(u["ffw"], x, llc_edge_index, N)
    enc_out = x
    y = _sage(params["dec"]["embed"], ast_x, ast_x, ast_edge_index, N)
    for u in params["dec"]["units"]:
        y = _attention(u["ast_att"], y, ast_edge_index, N)
        y = _tconv(u["cross"], y, enc_out, ast_edge_index, HEADS, HID, N)
        y = _gln(u["norm"], _lin(u["cat"], y))
        y = _ffw(u["ffw"], y, ast_edge_index, N)
    new_node = jax.nn.log_softmax(_lin(params["new_node"], y), axis=-1)
    node_sel = _sage(params["node_sel"], y, y, ast_edge_index, N)
    return new_node, node_sel


# speedup vs baseline: 3.9204x; 3.9204x over previous
import functools

import jax
import jax.numpy as jnp
from jax import lax
from jax.experimental import pallas as pl
from jax.experimental.pallas import tpu as pltpu
from jax.experimental.pallas import tpu_sc as plsc

N = 10000
E = 160000
HID = 256
HEADS = 4
WRK = 32
NEGF = -1e30

_DN = lax.GatherDimensionNumbers(
    offset_dims=(), collapsed_slice_dims=(0,), start_index_map=(0,))


def _rot(x, idx):
    return lax.gather(x, idx[:, None], _DN, (1,),
                      mode=lax.GatherScatterMode.PROMISE_IN_BOUNDS)


def _lane_sum(x):
    lane = lax.iota(jnp.int32, 16)
    for sh in (8, 4, 2, 1):
        x = x + _rot(x, (lane + sh) & 15)
    return x


def _mm_kernel(x_ref, w_ref, b_ref, o_ref):
    o_ref[...] = jnp.dot(x_ref[...], w_ref[...],
                         preferred_element_type=jnp.float32) + b_ref[...]


def matmul_bias(x, w, b, br=1000):
    n, din = x.shape
    dout = w.shape[1]
    return pl.pallas_call(
        _mm_kernel,
        grid=(n // br,),
        in_specs=[
            pl.BlockSpec((br, din), lambda i: (i, 0)),
            pl.BlockSpec((din, dout), lambda i: (0, 0)),
            pl.BlockSpec((dout,), lambda i: (0,)),
        ],
        out_specs=pl.BlockSpec((br, dout), lambda i: (i, 0)),
        out_shape=jax.ShapeDtypeStruct((n, dout), jnp.float32),
    )(x, w, b)


def _mm2_kernel(a_ref, occ_ref, x_ref, w1_ref, w2_ref, b_ref, o_ref):
    a = jnp.where(occ_ref[...] > 0, a_ref[...], 0.0)
    o_ref[...] = (jnp.dot(a, w1_ref[...], preferred_element_type=jnp.float32)
                  + jnp.dot(x_ref[...], w2_ref[...],
                            preferred_element_type=jnp.float32) + b_ref[...])


def sage_mm(agg, occ, x, w1, w2, b, br=1000):
    n, din = agg.shape
    dout = w1.shape[1]
    return pl.pallas_call(
        _mm2_kernel,
        grid=(n // br,),
        in_specs=[
            pl.BlockSpec((br, din), lambda i: (i, 0)),
            pl.BlockSpec((br, 1), lambda i: (i, 0)),
            pl.BlockSpec((br, x.shape[1]), lambda i: (i, 0)),
            pl.BlockSpec((din, dout), lambda i: (0, 0)),
            pl.BlockSpec((x.shape[1], dout), lambda i: (0, 0)),
            pl.BlockSpec((dout,), lambda i: (0,)),
        ],
        out_specs=pl.BlockSpec((br, dout), lambda i: (i, 0)),
        out_shape=jax.ShapeDtypeStruct((n, dout), jnp.float32),
    )(agg, occ, x, w1, w2, b)


def _cat_kernel(att_ref, occ_ref, skip_ref, w_ref, b_ref, o_ref):
    h = jnp.where(occ_ref[...] > 0, att_ref[...], 0.0) + skip_ref[...]
    o_ref[...] = jnp.dot(h, w_ref[...],
                         preferred_element_type=jnp.float32) + b_ref[...]


def cat_mm(att, occ, skip, w, b, br=1000):
    n, din = att.shape
    dout = w.shape[1]
    return pl.pallas_call(
        _cat_kernel,
        grid=(n // br,),
        in_specs=[
            pl.BlockSpec((br, din), lambda i: (i, 0)),
            pl.BlockSpec((br, 1), lambda i: (i, 0)),
            pl.BlockSpec((br, din), lambda i: (i, 0)),
            pl.BlockSpec((din, dout), lambda i: (0, 0)),
            pl.BlockSpec((dout,), lambda i: (0,)),
        ],
        out_specs=pl.BlockSpec((br, dout), lambda i: (i, 0)),
        out_shape=jax.ShapeDtypeStruct((n, dout), jnp.float32),
    )(att, occ, skip, w, b)


def _gln_kernel(x_ref, g_ref, b_ref, o_ref):
    x = x_ref[...]
    mu = jnp.mean(x)
    var = jnp.mean((x - mu) ** 2)
    o_ref[...] = (x - mu) / jnp.sqrt(var + 1e-5) * g_ref[...] + b_ref[...]


def gln(x, g, b):
    return pl.pallas_call(
        _gln_kernel,
        out_shape=jax.ShapeDtypeStruct(x.shape, jnp.float32),
    )(x, g, b)


def _lsm_kernel(x_ref, o_ref):
    x = x_ref[...]
    s = x - jnp.max(x, axis=1, keepdims=True)
    o_ref[...] = s - jnp.log(jnp.sum(jnp.exp(s), axis=1, keepdims=True))


def log_softmax_rows(x, br=1000):
    n, d = x.shape
    return pl.pallas_call(
        _lsm_kernel,
        grid=(n // br,),
        in_specs=[pl.BlockSpec((br, d), lambda i: (i, 0))],
        out_specs=pl.BlockSpec((br, d), lambda i: (i, 0)),
        out_shape=jax.ShapeDtypeStruct((n, d), jnp.float32),
    )(x)


def _attn_edge_sc(q, k, v, srcs, dsts, bounds):
    mesh = plsc.VectorSubcoreMesh(core_axis_name="c", subcore_axis_name="s")

    @functools.partial(
        pl.kernel, mesh=mesh,
        out_type=jax.ShapeDtypeStruct((N, HEADS * HID), jnp.float32),
        scratch_types=[
            pltpu.VMEM((48,), jnp.int32),
            pltpu.VMEM((16,), jnp.int32),
            pltpu.VMEM((16,), jnp.int32),
            pltpu.VMEM((32,), jnp.int32),
            pltpu.VMEM((16, 1024), jnp.float32),
            pltpu.VMEM((16, 1024), jnp.float32),
            pltpu.VMEM((16, 1024), jnp.float32),
            pltpu.VMEM((1024,), jnp.float32),
            pltpu.VMEM((1, 1024), jnp.float32),
            pltpu.SemaphoreType.DMA,
            pltpu.SemaphoreType.DMA,
            pltpu.SemaphoreType.DMA,
        ],
    )
    def kern(q_h, k_h, v_h, src_h, dst_h, bnd_h, out_h,
             bnd_v, sidx, didx16, didx, qr, kr, vr, acc, rowbuf, s1, s2, s3):
        wid = lax.axis_index("s") * 2 + lax.axis_index("c")
        pltpu.sync_copy(bnd_h, bnd_v)
        bb = bnd_v[pl.ds(wid, 16)]
        e_lo = bb[0]
        e_hi = bb[1]
        base0 = (e_lo // 16) * 16
        nch = jnp.maximum((e_hi - base0 + 15) // 16, 0)

        zvec = jnp.zeros((16,), jnp.float32)

        def zero_acc():
            for j in range(64):
                acc[pl.ds(j * 16, 16)] = zvec

        zero_acc()

        def flush(cur, dens):
            for h in range(HEADS):
                inv = 1.0 / (dens[h] + 1e-16)
                for j in range(16):
                    sl = pl.ds(h * 256 + j * 16, 16)
                    rowbuf[0, sl] = acc[sl] * inv
            pltpu.sync_copy(rowbuf, out_h.at[pl.ds(cur, 1)])
            zero_acc()

        NEG = jnp.float32(NEGF)

        def chunk_body(kc, carry):
            base = base0 + kc * 16
            pltpu.sync_copy(src_h.at[pl.ds(base, 16)], sidx)
            pltpu.sync_copy(dst_h.at[pl.ds(base, 16)], didx16)
            pltpu.sync_copy(dst_h.at[pl.ds(base, 32)], didx)
            ck = pltpu.async_copy(k_h.at[sidx], kr, s1)
            cv = pltpu.async_copy(v_h.at[sidx], vr, s2)
            cq = pltpu.async_copy(q_h.at[didx16], qr, s3)
            ck.wait()
            cv.wait()
            cq.wait()

            def edge_body(i, ec):
                cur, m0, m1, m2, m3, d0, d1, d2, d3 = ec
                m = [m0, m1, m2, m3]
                den = [d0, d1, d2, d3]
                e = base + i
                v_i = (e >= e_lo) & (e < e_hi)
                d_i = didx[pl.ds(i, 16)][0]
                is_new = v_i & (d_i != cur)

                @pl.when(is_new & (cur >= 0))
                def _():
                    flush(cur, den)

                cur = jnp.where(is_new, d_i, cur)
                m = [jnp.where(is_new, NEG, mh) for mh in m]
                den = [jnp.where(is_new, zvec, dh) for dh in den]
                newm = []
                newden = []
                for h in range(HEADS):
                    s = qr[i, pl.ds(h * 256, 16)] * kr[i, pl.ds(h * 256, 16)]
                    for j in range(1, 16):
                        sl = pl.ds(h * 256 + j * 16, 16)
                        s = s + qr[i, sl] * kr[i, sl]
                    lv = _lane_sum(s) * jnp.float32(1.0 / 16.0)
                    l_s = lv[0]
                    mh = m[h]
                    m_new = jnp.maximum(mh, l_s)
                    scale = jnp.exp(jnp.full((16,), mh - m_new, jnp.float32))
                    p = jnp.exp(lv - m_new)
                    dh_new = den[h] * scale + p

                    @pl.when(v_i & (l_s > mh) & (mh > NEG))
                    def _(h=h, scale=scale):
                        for j in range(16):
                            sl = pl.ds(h * 256 + j * 16, 16)
                            acc[sl] = acc[sl] * scale

                    @pl.when(v_i)
                    def _(h=h, p=p):
                        for j in range(16):
                            sl = pl.ds(h * 256 + j * 16, 16)
                            acc[sl] = acc[sl] + p * vr[i, sl]

                    newm.append(jnp.where(v_i, m_new, mh))
                    newden.append(jnp.where(v_i, dh_new, den[h]))
                return (cur, *newm, *newden)

            return lax.fori_loop(0, 16, edge_body, carry)

        init = (jnp.int32(-1), NEG, NEG, NEG, NEG, zvec, zvec, zvec, zvec)
        cur, m0, m1, m2, m3, d0, d1, d2, d3 = lax.fori_loop(
            0, nch, chunk_body, init)

        @pl.when(cur >= 0)
        def _():
            flush(cur, [d0, d1, d2, d3])

    return kern(q, k, v, srcs, dsts, bounds)


def _sage_edge_sc(x, srcs, dsts, bounds):
    mesh = plsc.VectorSubcoreMesh(core_axis_name="c", subcore_axis_name="s")

    @functools.partial(
        pl.kernel, mesh=mesh,
        out_type=jax.ShapeDtypeStruct((N, HID), jnp.float32),
        scratch_types=[
            pltpu.VMEM((48,), jnp.int32),
            pltpu.VMEM((16,), jnp.int32),
            pltpu.VMEM((32,), jnp.int32),
            pltpu.VMEM((16, HID), jnp.float32),
            pltpu.VMEM((HID,), jnp.float32),
            pltpu.VMEM((1, HID), jnp.float32),
            pltpu.SemaphoreType.DMA,
        ],
    )
    def kern(x_h, src_h, dst_h, bnd_h, out_h,
             bnd_v, sidx, didx, xr, acc, rowbuf, s1):
        wid = lax.axis_index("s") * 2 + lax.axis_index("c")
        pltpu.sync_copy(bnd_h, bnd_v)
        bb = bnd_v[pl.ds(wid, 16)]
        e_lo = bb[0]
        e_hi = bb[1]
        base0 = (e_lo // 16) * 16
        nch = jnp.maximum((e_hi - base0 + 15) // 16, 0)

        zvec = jnp.zeros((16,), jnp.float32)

        def zero_acc():
            for j in range(16):
                acc[pl.ds(j * 16, 16)] = zvec

        zero_acc()

        def flush(cur, cnt):
            inv = 1.0 / jnp.maximum(cnt, 1.0)
            for j in range(16):
                sl = pl.ds(j * 16, 16)
                rowbuf[0, sl] = acc[sl] * inv
            pltpu.sync_copy(rowbuf, out_h.at[pl.ds(cur, 1)])
            zero_acc()

        def chunk_body(kc, carry):
            base = base0 + kc * 16
            pltpu.sync_copy(src_h.at[pl.ds(base, 16)], sidx)
            pltpu.sync_copy(dst_h.at[pl.ds(base, 32)], didx)
            pltpu.async_copy(x_h.at[sidx], xr, s1).wait()

            def edge_body(i, ec):
                cur, cnt = ec
                e = base + i
                v_i = (e >= e_lo) & (e < e_hi)
                d_i = didx[pl.ds(i, 16)][0]
                is_new = v_i & (d_i != cur)

                @pl.when(is_new & (cur >= 0))
                def _():
                    flush(cur, cnt)

                cur = jnp.where(is_new, d_i, cur)
                cnt = jnp.where(is_new, zvec, cnt)

                @pl.when(v_i)
                def _():
                    for j in range(16):
                        sl = pl.ds(j * 16, 16)
                        acc[sl] = acc[sl] + xr[i, sl]

                cnt = jnp.where(v_i, cnt + 1.0, cnt)
                return (cur, cnt)

            return lax.fori_loop(0, 16, edge_body, carry)

        cur, cnt = lax.fori_loop(0, nch, chunk_body, (jnp.int32(-1), zvec))

        @pl.when(cur >= 0)
        def _():
            flush(cur, cnt)

    return kern(x, srcs, dsts, bounds)


def _prep(ei):
    src = ei[0].astype(jnp.int32)
    dst = ei[1].astype(jnp.int32)
    perm = jnp.argsort(dst)
    dst_s = dst[perm]
    src_s = src[perm]
    tgt = (jnp.arange(1, WRK) * E) // WRK
    dvals = dst_s[tgt]
    inner = jnp.searchsorted(dst_s, dvals, side='left').astype(jnp.int32)
    bounds = jnp.concatenate([
        jnp.zeros((1,), jnp.int32), inner,
        jnp.full((16,), E, jnp.int32)])
    src_p = jnp.concatenate([src_s, jnp.zeros((32,), jnp.int32)])
    dst_p = jnp.concatenate([dst_s, jnp.zeros((32,), jnp.int32)])
    rp = jnp.searchsorted(dst_s, jnp.arange(N + 1, dtype=jnp.int32),
                          side='left')
    occ = (rp[1:] > rp[:-1]).astype(jnp.float32)[:, None]
    return src_p, dst_p, bounds, occ


def _tconv_apply(p, x_src, x_dst, g):
    q = matmul_bias(x_dst, p["q"]["W"], p["q"]["b"])
    k = matmul_bias(x_src, p["k"]["W"], p["k"]["b"])
    v = matmul_bias(x_src, p["v"]["W"], p["v"]["b"])
    skip = matmul_bias(x_dst, p["skip"]["W"], p["skip"]["b"])
    att = _attn_edge_sc(q, k, v, g[0], g[1], g[2])
    return att, skip


def _attention_apply(p, x, g):
    att, skip = _tconv_apply(p["att"], x, x, g)
    h = cat_mm(att, g[3], skip, p["cat"]["W"], p["cat"]["b"])
    return gln(h, p["norm"]["g"], p["norm"]["b"])


def _ffw_apply(p, x, g):
    agg = _sage_edge_sc(x, g[0], g[1], g[2])
    o = sage_mm(agg, g[3], x, p["sage"]["l"]["W"], p["sage"]["Wr"],
                p["sage"]["l"]["b"])
    return gln(o, p["norm"]["g"], p["norm"]["b"])


def _sage_apply(p, x_src, x_dst, g):
    agg = _sage_edge_sc(x_src, g[0], g[1], g[2])
    return sage_mm(agg, g[3], x_dst, p["l"]["W"], p["Wr"], p["l"]["b"])


def kernel(ast_x, llc_x, params, ast_edge_index, llc_edge_index):
    ga = _prep(ast_edge_index)
    gl = _prep(llc_edge_index)

    x = _sage_apply(params["enc"]["embed"], llc_x, llc_x, gl)
    for u in params["enc"]["units"]:
        x = _attention_apply(u["att"], x, gl)
        x = _ffw_apply(u["ffw"], x, gl)
    enc_out = x

    y = _sage_apply(params["dec"]["embed"], ast_x, ast_x, ga)
    for u in params["dec"]["units"]:
        y = _attention_apply(u["ast_att"], y, ga)
        att, skip = _tconv_apply(u["cross"], y, enc_out, ga)
        h = cat_mm(att, ga[3], skip, u["cat"]["W"], u["cat"]["b"])
        y = gln(h, u["norm"]["g"], u["norm"]["b"])
        y = _ffw_apply(u["ffw"], y, ga)

    nn = matmul_bias(y, params["new_node"]["W"], params["new_node"]["b"])
    new_node = log_softmax_rows(nn)

    nsp = params["node_sel"]
    w1p = jnp.pad(nsp["l"]["W"], ((0, 0), (0, 7)))
    b1p = jnp.pad(nsp["l"]["b"], (0, 7))
    w2p = jnp.pad(nsp["Wr"], ((0, 0), (0, 7)))
    agg = _sage_edge_sc(y, ga[0], ga[1], ga[2])
    node_sel = sage_mm(agg, ga[3], y, w1p, w2p, b1p)[:, :1]
    return new_node, node_sel
